# Initial kernel scaffold; baseline (speedup 1.0000x reference)
import jax
import jax.numpy as jnp
from jax.experimental import pallas as pl


def kernel(x, edge_index, edge_attr, w1, r1, b1, g1, be1, w2, r2, b2, g2, be2, w3, r3, b3, g3, be3, w4, r4, b4):
    raise NotImplementedError("write your pallas kernel here")


# speedup vs baseline: 4.4245x; 4.4245x over previous
import functools

import jax
import jax.numpy as jnp
from jax import lax
from jax.experimental import pallas as pl
from jax.experimental.pallas import tpu as pltpu
from jax.experimental.pallas import tpu_sc as plsc

N = 10000
E = 160000
BE = 2000
KTOT = 16


def _bd_body(ea_ref, bd_ref):
    t = ea_ref[...]
    t0 = t[:, 0:1]
    t1 = t[:, 1:2]

    def b4(tt):
        c0 = (1.0 - tt) ** 3 / 6.0
        c1 = (3.0 * tt ** 3 - 6.0 * tt ** 2 + 4.0) / 6.0
        c2 = (-3.0 * tt ** 3 + 3.0 * tt ** 2 + 3.0 * tt + 1.0) / 6.0
        c3 = tt ** 3 / 6.0
        return jnp.concatenate([c0, c1, c2, c3], axis=1)

    b0 = b4(t0)
    b1 = b4(t1)
    bd = jnp.concatenate([b0 * b1[:, k:k + 1] for k in range(4)], axis=1)
    bd_ref[...] = bd


def _compute_bd(edge_attr):
    return pl.pallas_call(
        _bd_body,
        grid=(E // BE,),
        in_specs=[pl.BlockSpec((BE, 2), lambda i: (i, 0))],
        out_specs=pl.BlockSpec((BE, KTOT), lambda i: (i, 0)),
        out_shape=jax.ShapeDtypeStruct((E, KTOT), jnp.float32),
    )(edge_attr)


def _edge_body(xj_ref, bd_ref, w_ref, msg_ref, *, fin):
    xj = xj_ref[:, :fin]
    bd = bd_ref[...]
    acc = None
    for k in range(KTOT):
        part = jnp.dot(xj * bd[:, k:k + 1], w_ref[k],
                       preferred_element_type=jnp.float32)
        acc = part if acc is None else acc + part
    msg_ref[...] = jnp.pad(acc, ((0, 0), (0, 128 - acc.shape[1])))


def _edge_msgs(xj, bd, w):
    fin, fout = w.shape[1], w.shape[2]
    return pl.pallas_call(
        functools.partial(_edge_body, fin=fin),
        grid=(E // BE,),
        in_specs=[
            pl.BlockSpec((BE, 128), lambda i: (i, 0)),
            pl.BlockSpec((BE, KTOT), lambda i: (i, 0)),
            pl.BlockSpec((KTOT, fin, fout), lambda i: (0, 0, 0)),
        ],
        out_specs=pl.BlockSpec((BE, 128), lambda i: (i, 0)),
        out_shape=jax.ShapeDtypeStruct((E, 128), jnp.float32),
    )(xj, bd, w)


def _node_body(agg_ref, x_ref, root_ref, bias_ref, g_ref, be_ref, out_ref,
               *, act, fin, fout):
    agg = agg_ref[:N, :fout]
    agg = jnp.where(jnp.isfinite(agg), agg, 0.0)
    z = agg + jnp.dot(x_ref[:, :fin], root_ref[...],
                      preferred_element_type=jnp.float32) + bias_ref[...]
    if g_ref is not None:
        m = jnp.mean(z, axis=0, keepdims=True)
        v = jnp.mean((z - m) ** 2, axis=0, keepdims=True)
        z = g_ref[...] * (z - m) * jax.lax.rsqrt(v + 1e-5) + be_ref[...]
    if act == "elu":
        z = jnp.where(z > 0, z, jnp.exp(z) - 1.0)
    out_ref[...] = jnp.pad(z, ((0, 0), (0, 128 - fout)))


def _node_update(agg, x, root, bias, g, be, act):
    fin, fout = root.shape
    args = [agg, x, root, bias.reshape(1, fout)]
    specs = [
        pl.BlockSpec((_NPAD, 128), lambda: (0, 0)),
        pl.BlockSpec((N, 128), lambda: (0, 0)),
        pl.BlockSpec((fin, fout), lambda: (0, 0)),
        pl.BlockSpec((1, fout), lambda: (0, 0)),
    ]
    if g is None:
        body = functools.partial(_node_body, g_ref=None, be_ref=None, act=act,
                                 fin=fin, fout=fout)
    else:
        args += [g.reshape(1, fout), be.reshape(1, fout)]
        specs += [pl.BlockSpec((1, fout), lambda: (0, 0)),
                  pl.BlockSpec((1, fout), lambda: (0, 0))]
        body = functools.partial(_node_body, act=act, fin=fin, fout=fout)
    return pl.pallas_call(
        body,
        in_specs=specs,
        out_specs=pl.BlockSpec((N, 128), lambda: (0, 0)),
        out_shape=jax.ShapeDtypeStruct((N, 128), jnp.float32),
    )(*args)


def _msg4_body(xj_ref, bd_ref, w4t_ref, out_ref):
    y = jnp.dot(xj_ref[...], w4t_ref[...], preferred_element_type=jnp.float32)
    val = jnp.sum(y * bd_ref[...], axis=1, keepdims=True)
    out_ref[...] = jnp.broadcast_to(val, (val.shape[0], 128))


def _msg4(xj4, bd, w4t):
    return pl.pallas_call(
        _msg4_body,
        grid=(E // BE,),
        in_specs=[pl.BlockSpec((BE, 128), lambda i: (i, 0)),
                  pl.BlockSpec((BE, KTOT), lambda i: (i, 0)),
                  pl.BlockSpec((128, KTOT), lambda i: (0, 0))],
        out_specs=pl.BlockSpec((BE, 128), lambda i: (i, 0)),
        out_shape=jax.ShapeDtypeStruct((E, 128), jnp.float32),
    )(xj4, bd, w4t)


def _out4_body(agg_ref, h_ref, r4_ref, b_ref, out_ref):
    agg = agg_ref[:N, 0:1]
    agg = jnp.where(jnp.isfinite(agg), agg, 0.0)
    z = agg + jnp.dot(h_ref[...], r4_ref[...],
                      preferred_element_type=jnp.float32) + b_ref[...]
    z = jnp.where(z > 0, z, jnp.exp(z) - 1.0)
    out_ref[...] = jax.nn.sigmoid(z)


def _out4(agg4, h3, r4, b4):
    return pl.pallas_call(
        _out4_body,
        in_specs=[pl.BlockSpec((_NPAD, 128), lambda: (0, 0)),
                  pl.BlockSpec((N, 128), lambda: (0, 0)),
                  pl.BlockSpec((128, 1), lambda: (0, 0)),
                  pl.BlockSpec((1, 1), lambda: (0, 0))],
        out_specs=pl.BlockSpec((N, 1), lambda: (0, 0)),
        out_shape=jax.ShapeDtypeStruct((N, 1), jnp.float32),
    )(agg4, h3, r4, b4.reshape(1, 1))


_NC = 2
_NS = 16
_NW = _NC * _NS
_GC = 128


def _sc_mesh():
    return plsc.VectorSubcoreMesh(core_axis_name="c", subcore_axis_name="s")


def _lane_splat(vec, j):
    idx = jnp.full((16, 1), j, jnp.int32)
    return lax.gather(
        vec, idx,
        lax.GatherDimensionNumbers(offset_dims=(), collapsed_slice_dims=(0,),
                                   start_index_map=(0,)),
        (1,), mode=lax.GatherScatterMode.PROMISE_IN_BOUNDS)


def _sc_gather(table, idx, f):
    e = idx.shape[0]
    nchunks = e // _GC
    per_w = (nchunks + _NW - 1) // _NW

    @functools.partial(
        pl.kernel, mesh=_sc_mesh(),
        out_type=jax.ShapeDtypeStruct((e, f), jnp.float32),
        scratch_types=[pltpu.VMEM((_GC,), jnp.int32),
                       pltpu.VMEM((_GC, f), jnp.float32),
                       pltpu.SemaphoreType.DMA],
    )
    def k(table_hbm, idx_hbm, out_hbm, idx_v, rows_v, sem):
        wid = lax.axis_index("s") * _NC + lax.axis_index("c")

        def body(i, carry):
            c = wid + _NW * i

            @pl.when(c < nchunks)
            def _():
                base = c * _GC
                pltpu.sync_copy(idx_hbm.at[pl.ds(base, _GC)], idx_v)
                pltpu.async_copy(table_hbm.at[idx_v], rows_v, sem).wait()
                pltpu.sync_copy(rows_v, out_hbm.at[pl.ds(base, _GC)])

            return carry

        lax.fori_loop(0, per_w, body, 0)

    return k(table, idx)


_NPW = 320
_NPAD = _NW * _NPW
_SENT = _NPW
_CAPW = E + 128
_BCH = 640


def _sc_bucket(dst):
    nch = E // _BCH

    @functools.partial(
        pl.kernel, mesh=_sc_mesh(),
        out_type=jax.ShapeDtypeStruct((_NW, _CAPW), jnp.int32),
        scratch_types=[pltpu.VMEM((_BCH,), jnp.int32),
                       pltpu.VMEM((160,), jnp.int32)],
    )
    def k(dst_hbm, out_hbm, dv_buf, q):
        wid = lax.axis_index("s") * _NC + lax.axis_index("c")
        lo = wid * _NPW
        hi = jnp.minimum(lo + _NPW, N)
        iota = lax.iota(jnp.int32, 16)
        sent16 = jnp.full((16,), _SENT, jnp.int32)
        for i in range(10):
            q[pl.ds(16 * i, 16)] = sent16

        def chunk(c, carry):
            cur, written = carry
            pltpu.sync_copy(dst_hbm.at[pl.ds(c * _BCH, _BCH)], dv_buf)

            def group(gi, carry2):
                cur2, w2 = carry2
                dvg = dv_buf[pl.ds(16 * gi, 16)]
                mask = (dvg >= lo) & (dvg < hi)
                mi = mask.astype(jnp.int32)
                pref = plsc.cumsum(mi)
                cnt = jnp.sum(mi)
                ev = c * _BCH + 16 * gi + iota
                packed = ev * 512 + (dvg - lo)
                pos = cur2 - 1 + pref
                plsc.store_scatter(q, [pos], packed, mask=mask)
                cur2 = cur2 + cnt
                do_flush = cur2 >= 128

                @pl.when(do_flush)
                def _():
                    pltpu.sync_copy(
                        q.at[pl.ds(0, 128)],
                        out_hbm.at[wid, pl.ds(pl.multiple_of(w2, 128), 128)])
                    q[pl.ds(0, 16)] = q[pl.ds(128, 16)]
                    q[pl.ds(16, 16)] = q[pl.ds(144, 16)]

                cur2 = jnp.where(do_flush, cur2 - 128, cur2)
                w2 = jnp.where(do_flush, w2 + 128, w2)
                return cur2, w2

            return lax.fori_loop(0, _BCH // 16, group, (cur, written))

        cur, written = lax.fori_loop(0, nch, chunk, (0, 0))
        plsc.store_scatter(q, [cur + iota], sent16)
        total = cur + 16
        written = pl.multiple_of(written, 128)
        pltpu.sync_copy(q.at[pl.ds(0, 128)],
                        out_hbm.at[wid, pl.ds(written, 128)])

        @pl.when(total > 128)
        def _():
            q[pl.ds(0, 16)] = q[pl.ds(128, 16)]
            q[pl.ds(16, 16)] = q[pl.ds(144, 16)]
            pltpu.sync_copy(q.at[pl.ds(0, 128)],
                            out_hbm.at[wid, pl.ds(written + 128, 128)])

    return k(dst)


def _sc_scatter_max(msg, packed, ncg):
    maxch = _CAPW // 128

    @functools.partial(
        pl.kernel, mesh=_sc_mesh(),
        out_type=jax.ShapeDtypeStruct((_NPAD, 128), jnp.float32),
        scratch_types=[pltpu.VMEM((328, 128), jnp.float32),
                       pltpu.VMEM((128, 128), jnp.float32),
                       pltpu.VMEM((128,), jnp.int32),
                       pltpu.VMEM((128,), jnp.int32),
                       pltpu.SemaphoreType.DMA],
    )
    def k(msg_hbm, pk_hbm, agg_hbm, slab, rows_v, ids_v, dr_v, sem):
        wid = lax.axis_index("s") * _NC + lax.axis_index("c")
        ninf16 = jnp.full((16,), -jnp.inf, jnp.float32)
        iota = lax.iota(jnp.int32, 16)

        def init(i, carry):
            row = jnp.full((16,), i, jnp.int32)
            for g in range(8):
                plsc.store_scatter(slab, [row, 16 * g + iota], ninf16)
            return carry

        lax.fori_loop(0, _SENT + 1, init, 0)

        def cond(carry):
            c, done = carry
            return jnp.logical_not(done) & (c < maxch)

        def chunk(carry):
            c, done = carry
            pltpu.sync_copy(pk_hbm.at[wid, pl.ds(c * 128, 128)], ids_v)
            for g in range(8):
                pk = ids_v[pl.ds(16 * g, 16)]
                dr_v[pl.ds(16 * g, 16)] = jnp.minimum(pk & 511, _SENT)
                ids_v[pl.ds(16 * g, 16)] = pk >> 9
            pltpu.async_copy(msg_hbm.at[ids_v], rows_v, sem).wait()

            def group(gi, carry2):
                dvg = dr_v[pl.ds(16 * gi, 16)]
                for j in range(16):
                    djv = _lane_splat(dvg, j)
                    ej = jnp.full((16,), 16 * gi + j, jnp.int32)
                    for gc in range(ncg):
                        colv = 16 * gc + iota
                        cur = plsc.load_gather(slab, [djv, colv])
                        r = plsc.load_gather(rows_v, [ej, colv])
                        plsc.store_scatter(slab, [djv, colv],
                                           jnp.maximum(cur, r))
                return jnp.maximum(carry2, jnp.max(dvg))

            mx = lax.fori_loop(0, 8, group, jnp.int32(-1))
            return c + 1, done | (mx >= _SENT)

        lax.while_loop(cond, chunk, (jnp.int32(0), jnp.bool_(False)))
        pltpu.sync_copy(slab.at[pl.ds(0, _NPW)],
                        agg_hbm.at[pl.ds(wid * _NPW, _NPW)])

    return k(msg, packed)


def kernel(x, edge_index, edge_attr, w1, r1, b1, g1, be1, w2, r2, b2, g2, be2,
           w3, r3, b3, g3, be3, w4, r4, b4):
    src, dst = edge_index[0], edge_index[1]
    bd = _compute_bd(edge_attr)


    h = x
    for (w, r, b, g, be, act) in (
            (w1, r1, b1, g1, be1, "elu"),
            (w2, r2, b2, g2, be2, "elu"),
            (w3, r3, b3, g3, be3, "none")):
        xj = _sc_gather(h, src, 128)
        msg = _edge_msgs(xj, bd, w)
        agg = jnp.pad(jax.ops.segment_max(msg[:, :w.shape[2]], dst, num_segments=N),
                      ((0, _NPAD - N), (0, 128 - w.shape[2])))
        h = _node_update(agg, h, r, b, g, be, act)

    w4t = w4[:, :, 0].T
    xj4 = _sc_gather(h, src, 128)
    msg4 = _msg4(xj4, bd, w4t)
    agg4 = jnp.pad(jax.ops.segment_max(msg4[:, :1], dst, num_segments=N),
                   ((0, _NPAD - N), (0, 127)))
    return _out4(agg4, h, r4, b4)


# speedup vs baseline: 4.6306x; 1.0466x over previous
import functools

import jax
import jax.numpy as jnp
from jax import lax
from jax.experimental import pallas as pl
from jax.experimental.pallas import tpu as pltpu
from jax.experimental.pallas import tpu_sc as plsc

N = 10000
E = 160000
BE = 2000
KTOT = 16


def _bd_body(ea_ref, bd_ref):
    t = ea_ref[...]
    t0 = t[:, 0:1]
    t1 = t[:, 1:2]

    def b4(tt):
        c0 = (1.0 - tt) ** 3 / 6.0
        c1 = (3.0 * tt ** 3 - 6.0 * tt ** 2 + 4.0) / 6.0
        c2 = (-3.0 * tt ** 3 + 3.0 * tt ** 2 + 3.0 * tt + 1.0) / 6.0
        c3 = tt ** 3 / 6.0
        return jnp.concatenate([c0, c1, c2, c3], axis=1)

    b0 = b4(t0)
    b1 = b4(t1)
    bd = jnp.concatenate([b0 * b1[:, k:k + 1] for k in range(4)], axis=1)
    bd_ref[...] = bd


def _compute_bd(edge_attr):
    return pl.pallas_call(
        _bd_body,
        grid=(E // BE,),
        in_specs=[pl.BlockSpec((BE, 2), lambda i: (i, 0))],
        out_specs=pl.BlockSpec((BE, KTOT), lambda i: (i, 0)),
        out_shape=jax.ShapeDtypeStruct((E, KTOT), jnp.float32),
    )(edge_attr)


def _edge_body(xj_ref, bd_ref, w_ref, msg_ref, *, fin):
    xj = xj_ref[:, :fin]
    bd = bd_ref[...]
    acc = None
    for k in range(KTOT):
        part = jnp.dot(xj * bd[:, k:k + 1], w_ref[k],
                       preferred_element_type=jnp.float32)
        acc = part if acc is None else acc + part
    msg_ref[...] = jnp.pad(acc, ((0, 0), (0, 128 - acc.shape[1])))


def _edge_msgs(xj, bd, w):
    fin, fout = w.shape[1], w.shape[2]
    return pl.pallas_call(
        functools.partial(_edge_body, fin=fin),
        grid=(E // BE,),
        in_specs=[
            pl.BlockSpec((BE, 128), lambda i: (i, 0)),
            pl.BlockSpec((BE, KTOT), lambda i: (i, 0)),
            pl.BlockSpec((KTOT, fin, fout), lambda i: (0, 0, 0)),
        ],
        out_specs=pl.BlockSpec((BE, 128), lambda i: (i, 0)),
        out_shape=jax.ShapeDtypeStruct((E, 128), jnp.float32),
    )(xj, bd, w)


def _node_body(agg_ref, x_ref, root_ref, bias_ref, g_ref, be_ref, out_ref,
               *, act, fin, fout):
    agg = agg_ref[:N, :fout]
    agg = jnp.where(jnp.isfinite(agg), agg, 0.0)
    z = agg + jnp.dot(x_ref[:, :fin], root_ref[...],
                      preferred_element_type=jnp.float32) + bias_ref[...]
    if g_ref is not None:
        m = jnp.mean(z, axis=0, keepdims=True)
        v = jnp.mean((z - m) ** 2, axis=0, keepdims=True)
        z = g_ref[...] * (z - m) * jax.lax.rsqrt(v + 1e-5) + be_ref[...]
    if act == "elu":
        z = jnp.where(z > 0, z, jnp.exp(z) - 1.0)
    out_ref[...] = jnp.pad(z, ((0, 0), (0, 128 - fout)))


def _node_update(agg, x, root, bias, g, be, act):
    fin, fout = root.shape
    args = [agg, x, root, bias.reshape(1, fout)]
    specs = [
        pl.BlockSpec((_NPAD, 128), lambda: (0, 0)),
        pl.BlockSpec((N, 128), lambda: (0, 0)),
        pl.BlockSpec((fin, fout), lambda: (0, 0)),
        pl.BlockSpec((1, fout), lambda: (0, 0)),
    ]
    if g is None:
        body = functools.partial(_node_body, g_ref=None, be_ref=None, act=act,
                                 fin=fin, fout=fout)
    else:
        args += [g.reshape(1, fout), be.reshape(1, fout)]
        specs += [pl.BlockSpec((1, fout), lambda: (0, 0)),
                  pl.BlockSpec((1, fout), lambda: (0, 0))]
        body = functools.partial(_node_body, act=act, fin=fin, fout=fout)
    return pl.pallas_call(
        body,
        in_specs=specs,
        out_specs=pl.BlockSpec((N, 128), lambda: (0, 0)),
        out_shape=jax.ShapeDtypeStruct((N, 128), jnp.float32),
    )(*args)


def _msg4_body(xj_ref, bd_ref, w4t_ref, out_ref):
    y = jnp.dot(xj_ref[...], w4t_ref[...], preferred_element_type=jnp.float32)
    val = jnp.sum(y * bd_ref[...], axis=1, keepdims=True)
    out_ref[...] = jnp.broadcast_to(val, (val.shape[0], 128))


def _msg4(xj4, bd, w4t):
    return pl.pallas_call(
        _msg4_body,
        grid=(E // BE,),
        in_specs=[pl.BlockSpec((BE, 128), lambda i: (i, 0)),
                  pl.BlockSpec((BE, KTOT), lambda i: (i, 0)),
                  pl.BlockSpec((128, KTOT), lambda i: (0, 0))],
        out_specs=pl.BlockSpec((BE, 128), lambda i: (i, 0)),
        out_shape=jax.ShapeDtypeStruct((E, 128), jnp.float32),
    )(xj4, bd, w4t)


def _out4_body(agg_ref, h_ref, r4_ref, b_ref, out_ref):
    agg = agg_ref[:N, 0:1]
    agg = jnp.where(jnp.isfinite(agg), agg, 0.0)
    z = agg + jnp.dot(h_ref[...], r4_ref[...],
                      preferred_element_type=jnp.float32) + b_ref[...]
    z = jnp.where(z > 0, z, jnp.exp(z) - 1.0)
    out_ref[...] = jax.nn.sigmoid(z)


def _out4(agg4, h3, r4, b4):
    return pl.pallas_call(
        _out4_body,
        in_specs=[pl.BlockSpec((_NPAD, 128), lambda: (0, 0)),
                  pl.BlockSpec((N, 128), lambda: (0, 0)),
                  pl.BlockSpec((128, 1), lambda: (0, 0)),
                  pl.BlockSpec((1, 1), lambda: (0, 0))],
        out_specs=pl.BlockSpec((N, 1), lambda: (0, 0)),
        out_shape=jax.ShapeDtypeStruct((N, 1), jnp.float32),
    )(agg4, h3, r4, b4.reshape(1, 1))


_NC = 2
_NS = 16
_NW = _NC * _NS
_GC = 128


def _sc_mesh():
    return plsc.VectorSubcoreMesh(core_axis_name="c", subcore_axis_name="s")


def _lane_splat(vec, j):
    idx = jnp.full((16, 1), j, jnp.int32)
    return lax.gather(
        vec, idx,
        lax.GatherDimensionNumbers(offset_dims=(), collapsed_slice_dims=(0,),
                                   start_index_map=(0,)),
        (1,), mode=lax.GatherScatterMode.PROMISE_IN_BOUNDS)


def _sc_gather(table, idx, f):
    e = idx.shape[0]
    nchunks = e // _GC
    per_w = (nchunks + _NW - 1) // _NW

    @functools.partial(
        pl.kernel, mesh=_sc_mesh(),
        out_type=jax.ShapeDtypeStruct((e, f), jnp.float32),
        scratch_types=[pltpu.VMEM((_GC,), jnp.int32),
                       pltpu.VMEM((_GC, f), jnp.float32),
                       pltpu.SemaphoreType.DMA],
    )
    def k(table_hbm, idx_hbm, out_hbm, idx_v, rows_v, sem):
        wid = lax.axis_index("s") * _NC + lax.axis_index("c")

        def body(i, carry):
            c = wid + _NW * i

            @pl.when(c < nchunks)
            def _():
                base = c * _GC
                pltpu.sync_copy(idx_hbm.at[pl.ds(base, _GC)], idx_v)
                pltpu.async_copy(table_hbm.at[idx_v], rows_v, sem).wait()
                pltpu.sync_copy(rows_v, out_hbm.at[pl.ds(base, _GC)])

            return carry

        lax.fori_loop(0, per_w, body, 0)

    return k(table, idx)


_NPW = 320
_NPAD = _NW * _NPW
_SENT = _NPW
_CAPW = E + 128
_BCH = 640


def _sc_bucket(dst):
    nch = E // _BCH

    @functools.partial(
        pl.kernel, mesh=_sc_mesh(),
        out_type=jax.ShapeDtypeStruct((_NW, _CAPW), jnp.int32),
        scratch_types=[pltpu.VMEM((_BCH,), jnp.int32),
                       pltpu.VMEM((160,), jnp.int32)],
    )
    def k(dst_hbm, out_hbm, dv_buf, q):
        wid = lax.axis_index("s") * _NC + lax.axis_index("c")
        lo = wid * _NPW
        hi = jnp.minimum(lo + _NPW, N)
        iota = lax.iota(jnp.int32, 16)
        sent16 = jnp.full((16,), _SENT, jnp.int32)
        for i in range(10):
            q[pl.ds(16 * i, 16)] = sent16

        def chunk(c, carry):
            cur, written = carry
            pltpu.sync_copy(dst_hbm.at[pl.ds(c * _BCH, _BCH)], dv_buf)

            def group(gi, carry2):
                cur2, w2 = carry2
                dvg = dv_buf[pl.ds(16 * gi, 16)]
                mask = (dvg >= lo) & (dvg < hi)
                mi = mask.astype(jnp.int32)
                pref = plsc.cumsum(mi)
                cnt = jnp.sum(mi)
                ev = c * _BCH + 16 * gi + iota
                packed = ev * 512 + (dvg - lo)
                pos = cur2 - 1 + pref
                plsc.store_scatter(q, [pos], packed, mask=mask)
                cur2 = cur2 + cnt
                do_flush = cur2 >= 128

                @pl.when(do_flush)
                def _():
                    pltpu.sync_copy(
                        q.at[pl.ds(0, 128)],
                        out_hbm.at[wid, pl.ds(pl.multiple_of(w2, 128), 128)])
                    q[pl.ds(0, 16)] = q[pl.ds(128, 16)]
                    q[pl.ds(16, 16)] = q[pl.ds(144, 16)]

                cur2 = jnp.where(do_flush, cur2 - 128, cur2)
                w2 = jnp.where(do_flush, w2 + 128, w2)
                return cur2, w2

            return lax.fori_loop(0, _BCH // 16, group, (cur, written))

        cur, written = lax.fori_loop(0, nch, chunk, (0, 0))
        plsc.store_scatter(q, [cur + iota], sent16)
        total = cur + 16
        written = pl.multiple_of(written, 128)
        pltpu.sync_copy(q.at[pl.ds(0, 128)],
                        out_hbm.at[wid, pl.ds(written, 128)])

        @pl.when(total > 128)
        def _():
            q[pl.ds(0, 16)] = q[pl.ds(128, 16)]
            q[pl.ds(16, 16)] = q[pl.ds(144, 16)]
            pltpu.sync_copy(q.at[pl.ds(0, 128)],
                            out_hbm.at[wid, pl.ds(written + 128, 128)])

    return k(dst)


def _sc_scatter_max(msg, packed, ncg):
    maxch = _CAPW // 128

    @functools.partial(
        pl.kernel, mesh=_sc_mesh(),
        out_type=jax.ShapeDtypeStruct((_NPAD, 128), jnp.float32),
        scratch_types=[pltpu.VMEM((328, 128), jnp.float32),
                       pltpu.VMEM((128, 128), jnp.float32),
                       pltpu.VMEM((128,), jnp.int32),
                       pltpu.VMEM((128,), jnp.int32),
                       pltpu.SemaphoreType.DMA],
    )
    def k(msg_hbm, pk_hbm, agg_hbm, slab, rows_v, ids_v, dr_v, sem):
        wid = lax.axis_index("s") * _NC + lax.axis_index("c")
        ninf16 = jnp.full((16,), -jnp.inf, jnp.float32)
        iota = lax.iota(jnp.int32, 16)

        def init(i, carry):
            row = jnp.full((16,), i, jnp.int32)
            for g in range(8):
                plsc.store_scatter(slab, [row, 16 * g + iota], ninf16)
            return carry

        lax.fori_loop(0, _SENT + 1, init, 0)

        def cond(carry):
            c, done = carry
            return jnp.logical_not(done) & (c < maxch)

        def chunk(carry):
            c, done = carry
            pltpu.sync_copy(pk_hbm.at[wid, pl.ds(c * 128, 128)], ids_v)
            for g in range(8):
                pk = ids_v[pl.ds(16 * g, 16)]
                dr_v[pl.ds(16 * g, 16)] = jnp.minimum(pk & 511, _SENT)
                ids_v[pl.ds(16 * g, 16)] = pk >> 9
            pltpu.async_copy(msg_hbm.at[ids_v], rows_v, sem).wait()

            def group(gi, carry2):
                dvg = dr_v[pl.ds(16 * gi, 16)]
                for j in range(16):
                    djv = _lane_splat(dvg, j)
                    ej = jnp.full((16,), 16 * gi + j, jnp.int32)
                    for gc in range(ncg):
                        colv = 16 * gc + iota
                        cur = plsc.load_gather(slab, [djv, colv])
                        r = plsc.load_gather(rows_v, [ej, colv])
                        plsc.store_scatter(slab, [djv, colv],
                                           jnp.maximum(cur, r))
                return jnp.maximum(carry2, jnp.max(dvg))

            mx = lax.fori_loop(0, 8, group, jnp.int32(-1))
            return c + 1, done | (mx >= _SENT)

        lax.while_loop(cond, chunk, (jnp.int32(0), jnp.bool_(False)))
        pltpu.sync_copy(slab.at[pl.ds(0, _NPW)],
                        agg_hbm.at[pl.ds(wid * _NPW, _NPW)])

    return k(msg, packed)


def kernel(x, edge_index, edge_attr, w1, r1, b1, g1, be1, w2, r2, b2, g2, be2,
           w3, r3, b3, g3, be3, w4, r4, b4):
    src, dst = edge_index[0], edge_index[1]
    order = jnp.argsort(dst)
    src = src[order]
    dst = jnp.sort(dst)
    bd = _compute_bd(edge_attr[order])


    h = x
    for (w, r, b, g, be, act) in (
            (w1, r1, b1, g1, be1, "elu"),
            (w2, r2, b2, g2, be2, "elu"),
            (w3, r3, b3, g3, be3, "none")):
        xj = _sc_gather(h, src, 128)
        msg = _edge_msgs(xj, bd, w)
        agg = jnp.pad(jax.ops.segment_max(msg[:, :w.shape[2]], dst, num_segments=N,
                                          indices_are_sorted=True),
                      ((0, _NPAD - N), (0, 128 - w.shape[2])))
        h = _node_update(agg, h, r, b, g, be, act)

    w4t = w4[:, :, 0].T
    xj4 = _sc_gather(h, src, 128)
    msg4 = _msg4(xj4, bd, w4t)
    agg4 = jnp.pad(jax.ops.segment_max(msg4[:, :1], dst, num_segments=N,
                                       indices_are_sorted=True),
                   ((0, _NPAD - N), (0, 127)))
    return _out4(agg4, h, r4, b4)
